# Initial kernel scaffold; baseline (speedup 1.0000x reference)
import jax
import jax.numpy as jnp
from jax.experimental import pallas as pl


def kernel(input_ids, token_type_ids, attention_mask, emb, W, b):
    raise NotImplementedError("write your pallas kernel here")


# speedup vs baseline: 3.4627x; 3.4627x over previous
import functools

import numpy as np
import jax
import jax.numpy as jnp
from jax import lax
from jax.experimental import pallas as pl
from jax.experimental.pallas import tpu as pltpu
from jax.experimental.pallas import tpu_sc as plsc

_B, _L, _V, _H, _O = 4096, 50, 100000, 128, 10000
_NC, _NS = 2, 16
_NW = _NC * _NS
_BPW = _B // _NW
_CB = 2
_ROWS = _CB * _L
_NCH = _BPW // _CB
_LANES = 16

_SIDX_HOST = np.asarray(
    (np.arange(_NS, dtype=np.int32)[:, None, None] * _BPW)
    + (np.arange(_NCH, dtype=np.int32)[None, :, None] * _CB)
    + (np.arange(_ROWS, dtype=np.int32)[None, None, :] // _L),
    dtype=np.int32,
)


def _sc_pool_body(ids_hbm, emb_hbm, sidx_hbm, out_hbm,
                  ids_v, sidx_v, rows0, rows1, acc_v, shared, sem0, sem1):
    c = lax.axis_index("c")
    s = lax.axis_index("s")
    w = c * _NS + s

    pltpu.sync_copy(ids_hbm.at[w], ids_v)
    pltpu.sync_copy(sidx_hbm.at[s], sidx_v)

    zero = jnp.zeros((_LANES,), jnp.float32)

    @pl.loop(0, _BPW)
    def _zero(i):
        for j in range(_H // _LANES):
            acc_v[i, pl.ds(j * _LANES, _LANES)] = zero

    pltpu.sync_copy(acc_v, shared.at[pl.ds(s * _BPW, _BPW)])

    pltpu.async_copy(emb_hbm.at[ids_v.at[0]], rows0, sem0)

    @pl.loop(0, _NCH, step=2)
    def _chunks(ch):
        pltpu.async_copy(emb_hbm.at[ids_v.at[ch + 1]], rows1, sem1)
        pltpu.make_async_copy(emb_hbm.at[ids_v.at[ch]], rows0, sem0).wait()
        pltpu.sync_copy(rows0, shared.at[sidx_v.at[ch]], add=True)

        @pl.when(ch + 2 < _NCH)
        def _():
            pltpu.async_copy(emb_hbm.at[ids_v.at[ch + 2]], rows0, sem0)

        pltpu.make_async_copy(emb_hbm.at[ids_v.at[ch + 1]], rows1, sem1).wait()
        pltpu.sync_copy(rows1, shared.at[sidx_v.at[ch + 1]], add=True)

    pltpu.sync_copy(shared.at[pl.ds(s * _BPW, _BPW)], acc_v)

    inv = jnp.full((_LANES,), 1.0 / _L, jnp.float32)

    @pl.loop(0, _BPW)
    def _act(i):
        for j in range(_H // _LANES):
            v = acc_v[i, pl.ds(j * _LANES, _LANES)]
            acc_v[i, pl.ds(j * _LANES, _LANES)] = jnp.maximum(v * inv, zero)

    pltpu.sync_copy(acc_v, out_hbm.at[w])


_sc_pool = pl.kernel(
    _sc_pool_body,
    out_type=jax.ShapeDtypeStruct((_NW, _BPW, _H), jnp.float32),
    mesh=plsc.VectorSubcoreMesh(core_axis_name="c", subcore_axis_name="s"),
    scratch_types=[
        pltpu.VMEM((_NCH, _ROWS), jnp.int32),
        pltpu.VMEM((_NCH, _ROWS), jnp.int32),
        pltpu.VMEM((_ROWS, _H), jnp.float32),
        pltpu.VMEM((_ROWS, _H), jnp.float32),
        pltpu.VMEM((_BPW, _H), jnp.float32),
        pltpu.VMEM_SHARED((_NS * _BPW, _H), jnp.float32),
        pltpu.SemaphoreType.DMA,
        pltpu.SemaphoreType.DMA,
    ],
)


_MB = 512
_NB = 2048
_GN = (_O + _NB - 1) // _NB


def _mm_body(x_ref, w_ref, b_ref, o_ref):
    o_ref[...] = lax.dot_general(
        x_ref[...], w_ref[...], (((1,), (1,)), ((), ())),
        preferred_element_type=jnp.float32,
    ) + b_ref[...]


def _matmul(x, W, b):
    return pl.pallas_call(
        _mm_body,
        grid=(_B // _MB, _GN),
        in_specs=[
            pl.BlockSpec((_MB, _H), lambda i, j: (i, 0)),
            pl.BlockSpec((_NB, _H), lambda i, j: (j, 0)),
            pl.BlockSpec((1, _NB), lambda i, j: (0, j)),
        ],
        out_specs=pl.BlockSpec((_MB, _NB), lambda i, j: (i, j)),
        out_shape=jax.ShapeDtypeStruct((_B, _O), jnp.float32),
    )(x, W, b.reshape(1, _O))


@jax.jit
def _impl(input_ids, emb, W, b):
    ids = input_ids.reshape(_NW, _NCH, _ROWS)
    pooled = _sc_pool(ids, emb, jnp.asarray(_SIDX_HOST)).reshape(_B, _H)
    return _matmul(pooled, W, b)


def kernel(input_ids, token_type_ids, attention_mask, emb, W, b):
    return _impl(input_ids, emb, W, b)


# speedup vs baseline: 3.6870x; 1.0648x over previous
import functools

import numpy as np
import jax
import jax.numpy as jnp
from jax import lax
from jax.experimental import pallas as pl
from jax.experimental.pallas import tpu as pltpu
from jax.experimental.pallas import tpu_sc as plsc

_B, _L, _V, _H, _O = 4096, 50, 100000, 128, 10000
_NC, _NS = 2, 16
_NW = _NC * _NS
_BPW = _B // _NW
_CB = 2
_ROWS = _CB * _L
_NCH = _BPW // _CB
_LANES = 16

_SIDX_HOST = np.asarray(
    (np.arange(_NS, dtype=np.int32)[:, None, None] * _BPW)
    + (np.arange(_NCH, dtype=np.int32)[None, :, None] * _CB)
    + (np.arange(_ROWS, dtype=np.int32)[None, None, :] // _L),
    dtype=np.int32,
)


def _sc_pool_body(ids_hbm, emb_hbm, sidx_hbm, out_hbm,
                  ids_v, sidx_v, rows0, rows1, acc_v, shared, sem0, sem1):
    c = lax.axis_index("c")
    s = lax.axis_index("s")
    w = c * _NS + s

    pltpu.sync_copy(ids_hbm.at[w], ids_v)
    pltpu.sync_copy(sidx_hbm.at[s], sidx_v)

    zero = jnp.zeros((_LANES,), jnp.float32)

    @pl.loop(0, _BPW)
    def _zero(i):
        for j in range(_H // _LANES):
            acc_v[i, pl.ds(j * _LANES, _LANES)] = zero

    pltpu.sync_copy(acc_v, shared.at[pl.ds(s * _BPW, _BPW)])

    pltpu.async_copy(emb_hbm.at[ids_v.at[0]], rows0, sem0)

    @pl.loop(0, _NCH, step=2)
    def _chunks(ch):
        pltpu.async_copy(emb_hbm.at[ids_v.at[ch + 1]], rows1, sem1)
        pltpu.make_async_copy(emb_hbm.at[ids_v.at[ch]], rows0, sem0).wait()
        pltpu.sync_copy(rows0, shared.at[sidx_v.at[ch]], add=True)

        @pl.when(ch + 2 < _NCH)
        def _():
            pltpu.async_copy(emb_hbm.at[ids_v.at[ch + 2]], rows0, sem0)

        pltpu.make_async_copy(emb_hbm.at[ids_v.at[ch + 1]], rows1, sem1).wait()
        pltpu.sync_copy(rows1, shared.at[sidx_v.at[ch + 1]], add=True)

    pltpu.sync_copy(shared.at[pl.ds(s * _BPW, _BPW)], acc_v)

    inv = jnp.full((_LANES,), 1.0 / _L, jnp.float32)

    @pl.loop(0, _BPW)
    def _act(i):
        for j in range(_H // _LANES):
            v = acc_v[i, pl.ds(j * _LANES, _LANES)]
            acc_v[i, pl.ds(j * _LANES, _LANES)] = jnp.maximum(v * inv, zero)

    pltpu.sync_copy(acc_v, out_hbm.at[w])


_sc_pool = pl.kernel(
    _sc_pool_body,
    out_type=jax.ShapeDtypeStruct((_NW, _BPW, _H), jnp.float32),
    mesh=plsc.VectorSubcoreMesh(core_axis_name="c", subcore_axis_name="s"),
    scratch_types=[
        pltpu.VMEM((_NCH, _ROWS), jnp.int32),
        pltpu.VMEM((_NCH, _ROWS), jnp.int32),
        pltpu.VMEM((_ROWS, _H), jnp.float32),
        pltpu.VMEM((_ROWS, _H), jnp.float32),
        pltpu.VMEM((_BPW, _H), jnp.float32),
        pltpu.VMEM_SHARED((_NS * _BPW, _H), jnp.float32),
        pltpu.SemaphoreType.DMA,
        pltpu.SemaphoreType.DMA,
    ],
)


_NB = 1024
_GN = (_O + _NB - 1) // _NB


def _mm_body(x_ref, w_ref, b_ref, o_ref):
    o_ref[...] = lax.dot_general(
        x_ref[...], w_ref[...], (((1,), (1,)), ((), ())),
        preferred_element_type=jnp.float32,
    ) + b_ref[...]


def _matmul(x, W, b):
    return pl.pallas_call(
        _mm_body,
        grid=(_GN,),
        in_specs=[
            pl.BlockSpec((_B, _H), lambda j: (0, 0)),
            pl.BlockSpec((_NB, _H), lambda j: (j, 0)),
            pl.BlockSpec((1, _NB), lambda j: (0, j)),
        ],
        out_specs=pl.BlockSpec((_B, _NB), lambda j: (0, j)),
        out_shape=jax.ShapeDtypeStruct((_B, _O), jnp.float32),
    )(x, W, b.reshape(1, _O))


@jax.jit
def _impl(input_ids, emb, W, b):
    ids = input_ids.reshape(_NW, _NCH, _ROWS)
    pooled = _sc_pool(ids, emb, jnp.asarray(_SIDX_HOST)).reshape(_B, _H)
    return _matmul(pooled, W, b)


def kernel(input_ids, token_type_ids, attention_mask, emb, W, b):
    return _impl(input_ids, emb, W, b)
